# Initial kernel scaffold; baseline (speedup 1.0000x reference)
import jax
import jax.numpy as jnp
from jax.experimental import pallas as pl


def kernel(x, edge_index, W):
    raise NotImplementedError("write your pallas kernel here")


# speedup vs baseline: 3.4889x; 3.4889x over previous
import functools

import jax
import jax.numpy as jnp
from jax import lax
from jax.experimental import pallas as pl
from jax.experimental.pallas import tpu as pltpu
from jax.experimental.pallas import tpu_sc as plsc

_NC = 2
_NS = 16
_CHUNK = 80


def _sc_aggregate(x0, x1, src, dst):
    n, dh = x0.shape
    e = src.shape[0]
    assert e % (_NS * _CHUNK) == 0, (e, _NS * _CHUNK)
    ept = e // _NS
    nchunk = ept // _CHUNK
    rpt = -(-n // (_NS * 8)) * 8
    np_ = rpt * _NS

    mesh = plsc.VectorSubcoreMesh(core_axis_name="c", subcore_axis_name="s")

    @functools.partial(
        pl.kernel,
        mesh=mesh,
        out_type=jax.ShapeDtypeStruct((_NC, np_, dh), jnp.float32),
        scratch_types=[
            pltpu.VMEM_SHARED((np_, dh), jnp.float32),
            pltpu.VMEM((rpt, dh), jnp.float32),
            pltpu.VMEM((_CHUNK,), jnp.int32),
            pltpu.VMEM((_CHUNK,), jnp.int32),
            pltpu.VMEM((_CHUNK, dh), jnp.float32),
            pltpu.SemaphoreType.DMA,
        ],
        compiler_params=pltpu.CompilerParams(use_tc_tiling_on_sc=False),
    )
    def agg(x0_hbm, x1_hbm, src_hbm, dst_hbm, out_hbm,
            acc, zbuf, sidx, didx, rows, sem):
        cid = lax.axis_index("c")
        sid = lax.axis_index("s")
        row0 = pl.multiple_of(sid * rpt, rpt)

        zv = jnp.zeros((16,), jnp.float32)
        lanes = dh // 16

        def zero_body(i, carry):
            zbuf[i // lanes, pl.ds((i % lanes) * 16, 16)] = zv
            return carry

        lax.fori_loop(0, rpt * lanes, zero_body, 0)
        pltpu.sync_copy(zbuf, acc.at[pl.ds(row0, rpt)])
        plsc.subcore_barrier()

        ebase = sid * ept

        def edge_body(j, carry):
            off = pl.multiple_of(ebase + j * _CHUNK, _CHUNK)
            pltpu.sync_copy(src_hbm.at[pl.ds(off, _CHUNK)], sidx)
            pltpu.sync_copy(dst_hbm.at[pl.ds(off, _CHUNK)], didx)

            @pl.when(cid == 0)
            def _():
                pltpu.async_copy(x0_hbm.at[sidx], rows, sem).wait()

            @pl.when(cid == 1)
            def _():
                pltpu.async_copy(x1_hbm.at[sidx], rows, sem).wait()

            pltpu.sync_copy(rows, acc.at[didx], add=True)
            return carry

        lax.fori_loop(0, nchunk, edge_body, 0)
        plsc.subcore_barrier()

        pltpu.sync_copy(acc.at[pl.ds(row0, rpt)], zbuf)
        pltpu.sync_copy(zbuf, out_hbm.at[cid, pl.ds(row0, rpt)])

    return agg(x0, x1, src, dst)


def _tc_finish(agg, w, n):
    dh = agg.shape[2]
    dout = w.shape[1]
    br = 1000
    while n % br:
        br -= 8

    def body(p_ref, w_ref, o_ref):
        o_ref[...] = (
            jnp.dot(p_ref[0], w_ref[:dh], preferred_element_type=jnp.float32)
            + jnp.dot(p_ref[1], w_ref[dh:], preferred_element_type=jnp.float32)
        )

    return pl.pallas_call(
        body,
        grid=(n // br,),
        in_specs=[
            pl.BlockSpec((_NC, br, dh), lambda i: (0, i, 0)),
            pl.BlockSpec((2 * dh, dout), lambda i: (0, 0)),
        ],
        out_specs=pl.BlockSpec((br, dout), lambda i: (i, 0)),
        out_shape=jax.ShapeDtypeStruct((n, dout), jnp.float32),
    )(agg, w)


def kernel(x, edge_index, W):
    dst = edge_index[0].astype(jnp.int32)
    src = edge_index[1].astype(jnp.int32)
    dh = x.shape[1] // 2
    x0 = x[:, :dh]
    x1 = x[:, dh:]
    agg = _sc_aggregate(x0, x1, src, dst)
    return _tc_finish(agg, W, x.shape[0])


# speedup vs baseline: 6.2352x; 1.7871x over previous
import functools

import jax
import jax.numpy as jnp
from jax import lax
from jax.experimental import pallas as pl
from jax.experimental.pallas import tpu as pltpu
from jax.experimental.pallas import tpu_sc as plsc

_NC = 2
_NS = 16
_CHUNK = 80


def _sc_aggregate(x01, src_ab, dst_t, n, dh):
    cpt = src_ab.shape[2]
    assert cpt % 2 == 0
    ng = cpt // 2
    rpt = -(-n // (_NS * 8)) * 8
    np_ = rpt * _NS
    zrows = rpt - 8
    rrows = rpt - zrows

    mesh = plsc.VectorSubcoreMesh(core_axis_name="c", subcore_axis_name="s")

    @functools.partial(
        pl.kernel,
        mesh=mesh,
        out_type=jax.ShapeDtypeStruct((_NC, np_, dh), jnp.float32),
        scratch_types=[
            pltpu.VMEM_SHARED((np_, dh), jnp.float32),
            pltpu.VMEM((zrows, dh), jnp.float32),
            pltpu.VMEM((cpt, _CHUNK), jnp.int32),
            pltpu.VMEM((cpt, _CHUNK), jnp.int32),
            pltpu.VMEM((_CHUNK, dh), jnp.float32),
            pltpu.VMEM((_CHUNK, dh), jnp.float32),
            pltpu.SemaphoreType.DMA,
            pltpu.SemaphoreType.DMA,
            pltpu.SemaphoreType.DMA,
            pltpu.SemaphoreType.DMA,
        ],
        compiler_params=pltpu.CompilerParams(use_tc_tiling_on_sc=False),
    )
    def agg(x01_hbm, src_hbm, dst_hbm, out_hbm,
            acc, zbuf, src_v, dst_v, rows0, rows1, sg0, sg1, ss0, ss1):
        cid = lax.axis_index("c")
        sid = lax.axis_index("s")
        row0 = pl.multiple_of(sid * rpt, rpt)

        zv = jnp.zeros((16,), jnp.float32)
        lanes = dh // 16

        def zero_body(i, carry):
            zbuf[i // lanes, pl.ds((i % lanes) * 16, 16)] = zv
            return carry

        lax.fori_loop(0, zrows * lanes, zero_body, 0)
        pltpu.sync_copy(zbuf, acc.at[pl.ds(row0, zrows)])
        pltpu.sync_copy(zbuf.at[pl.ds(0, rrows)],
                        acc.at[pl.ds(row0 + zrows, rrows)])

        pltpu.sync_copy(src_hbm.at[cid, sid], src_v)
        pltpu.sync_copy(dst_hbm.at[sid], dst_v)
        plsc.subcore_barrier()

        def g_start(j, buf, sem):
            pltpu.async_copy(x01_hbm.at[src_v.at[j]], buf, sem)

        def g_wait(j, buf, sem):
            pltpu.make_async_copy(x01_hbm.at[src_v.at[j]], buf, sem).wait()

        def s_start(j, buf, sem):
            pltpu.async_copy(buf, acc.at[dst_v.at[j]], sem, add=True)

        def s_wait(j, buf, sem):
            pltpu.make_async_copy(buf, acc.at[dst_v.at[j]], sem).wait()

        g_start(0, rows0, sg0)

        def body(g, carry):
            j0 = g * 2
            j1 = j0 + 1
            g_wait(j0, rows0, sg0)

            @pl.when(g > 0)
            def _():
                s_wait(j1 - 2, rows1, ss1)

            g_start(j1, rows1, sg1)
            s_start(j0, rows0, ss0)
            g_wait(j1, rows1, sg1)
            s_wait(j0, rows0, ss0)

            @pl.when(g < ng - 1)
            def _():
                g_start(j0 + 2, rows0, sg0)

            s_start(j1, rows1, ss1)
            return carry

        lax.fori_loop(0, ng, body, 0)
        s_wait(cpt - 1, rows1, ss1)
        plsc.subcore_barrier()

        pltpu.sync_copy(acc.at[pl.ds(row0, zrows)], zbuf)
        pltpu.sync_copy(zbuf, out_hbm.at[cid, pl.ds(row0, zrows)])
        pltpu.sync_copy(acc.at[pl.ds(row0 + zrows, rrows)],
                        zbuf.at[pl.ds(0, rrows)])
        pltpu.sync_copy(zbuf.at[pl.ds(0, rrows)],
                        out_hbm.at[cid, pl.ds(row0 + zrows, rrows)])

    return agg(x01, src_ab, dst_t)


def _tc_finish(agg, w, n):
    dh = agg.shape[2]
    dout = w.shape[1]
    br = 1000
    while n % br:
        br -= 8

    def body(p_ref, w_ref, o_ref):
        o_ref[...] = (
            jnp.dot(p_ref[0], w_ref[:dh], preferred_element_type=jnp.float32)
            + jnp.dot(p_ref[1], w_ref[dh:], preferred_element_type=jnp.float32)
        )

    return pl.pallas_call(
        body,
        grid=(n // br,),
        in_specs=[
            pl.BlockSpec((_NC, br, dh), lambda i: (0, i, 0)),
            pl.BlockSpec((2 * dh, dout), lambda i: (0, 0)),
        ],
        out_specs=pl.BlockSpec((br, dout), lambda i: (i, 0)),
        out_shape=jax.ShapeDtypeStruct((n, dout), jnp.float32),
    )(agg, w)


def kernel(x, edge_index, W):
    n, d = x.shape
    dh = d // 2
    e = edge_index.shape[1]
    cpt = e // (_NS * _CHUNK)
    dst = edge_index[0].astype(jnp.int32)
    src = edge_index[1].astype(jnp.int32)
    x01 = jnp.concatenate([x[:, :dh], x[:, dh:]], axis=0)
    src_ab = jnp.stack([src, src + n]).reshape(_NC, _NS, cpt, _CHUNK)
    dst_t = dst.reshape(_NS, cpt, _CHUNK)
    agg = _sc_aggregate(x01, src_ab, dst_t, n, dh)
    return _tc_finish(agg, W, n)


# speedup vs baseline: 12.6063x; 2.0218x over previous
import functools

import jax
import jax.numpy as jnp
from jax import lax
from jax.experimental import pallas as pl
from jax.experimental.pallas import tpu as pltpu
from jax.experimental.pallas import tpu_sc as plsc

_NC = 2
_NS = 16
_CHUNK = 80
_NBUF = 4


def _sc_aggregate(x01, src_t, dst_t, n, dh):
    cpt = src_t.shape[1]
    tail = cpt % _NBUF
    ng = cpt // _NBUF
    rpt = -(-n // (_NS * 8)) * 8
    np_ = rpt * _NS
    zrows = 320
    lanes = dh // 16
    cl = _CHUNK // 16

    mesh = plsc.VectorSubcoreMesh(core_axis_name="c", subcore_axis_name="s")

    @functools.partial(
        pl.kernel,
        mesh=mesh,
        out_type=jax.ShapeDtypeStruct((_NC, np_, dh), jnp.float32),
        scratch_types=[
            pltpu.VMEM_SHARED((np_, dh), jnp.float32),
            pltpu.VMEM((zrows, dh), jnp.float32),
            pltpu.VMEM((cpt, _CHUNK), jnp.int32),
            pltpu.VMEM((cpt, _CHUNK), jnp.int32),
            [pltpu.VMEM((_CHUNK, dh), jnp.float32) for _ in range(_NBUF)],
            [pltpu.SemaphoreType.DMA for _ in range(_NBUF)],
            pltpu.SemaphoreType.DMA,
            pltpu.SemaphoreType.DMA,
        ],
        compiler_params=pltpu.CompilerParams(use_tc_tiling_on_sc=False),
    )
    def agg(x01_hbm, src_hbm, dst_hbm, out_hbm,
            acc, zbuf, src_v, dst_v, rows, sems, si_sem, di_sem):
        cid = lax.axis_index("c")
        sid = lax.axis_index("s")
        row0 = pl.multiple_of(sid * rpt, rpt)

        pltpu.async_copy(src_hbm.at[sid], src_v, si_sem)
        pltpu.async_copy(dst_hbm.at[sid], dst_v, di_sem)

        zv = jnp.zeros((16,), jnp.float32)

        def zero_body(i, carry):
            zbuf[i // lanes, pl.ds((i % lanes) * 16, 16)] = zv
            return carry

        lax.fori_loop(0, zrows * lanes, zero_body, 0)
        done = 0
        while done < rpt:
            step = min(zrows, rpt - done)
            pltpu.sync_copy(zbuf.at[pl.ds(0, step)],
                            acc.at[pl.ds(row0 + done, step)])
            done += step

        pltpu.make_async_copy(src_hbm.at[sid], src_v, si_sem).wait()
        pltpu.make_async_copy(dst_hbm.at[sid], dst_v, di_sem).wait()
        plsc.subcore_barrier()

        def adjust(j):
            for c in range(cl):
                v = src_v[j, pl.ds(c * 16, 16)]
                src_v[j, pl.ds(c * 16, 16)] = v + v + cid

        def g_start(j, b):
            pltpu.async_copy(x01_hbm.at[src_v.at[j]], rows[b], sems[b])

        def g_wait(j, b):
            pltpu.make_async_copy(x01_hbm.at[src_v.at[j]], rows[b],
                                  sems[b]).wait()

        def s_start(j, b):
            pltpu.async_copy(rows[b], acc.at[dst_v.at[j]], sems[b], add=True)

        def s_wait(j, b):
            pltpu.make_async_copy(rows[b], acc.at[dst_v.at[j]],
                                  sems[b]).wait()

        for j in range(_NBUF - 1):
            adjust(j)
            g_start(j, j)

        def body(g, carry):
            jb = g * _NBUF
            for b in range(_NBUF):
                j = jb + b
                g_wait(j, b)
                s_start(j, b)

                if b == 0:
                    @pl.when(g > 0)
                    def _():
                        s_wait(j - 1, _NBUF - 1)
                else:
                    s_wait(j - 1, b - 1)

                @pl.when(j + _NBUF - 1 < cpt)
                def _():
                    adjust(j + _NBUF - 1)
                    g_start(j + _NBUF - 1, (b - 1) % _NBUF)

            return carry

        lax.fori_loop(0, ng, body, 0)
        for j in range(cpt - tail, cpt):
            b = j % _NBUF
            g_wait(j, b)
            s_start(j, b)
            s_wait(j - 1, (b - 1) % _NBUF)
        s_wait(cpt - 1, (cpt - 1) % _NBUF)
        plsc.subcore_barrier()

        done = 0
        while done < rpt:
            step = min(zrows, rpt - done)
            pltpu.sync_copy(acc.at[pl.ds(row0 + done, step)],
                            zbuf.at[pl.ds(0, step)])
            pltpu.sync_copy(zbuf.at[pl.ds(0, step)],
                            out_hbm.at[cid, pl.ds(row0 + done, step)])
            done += step

    return agg(x01, src_t, dst_t)


def _tc_finish(agg, w, n):
    dh = agg.shape[2]
    dout = w.shape[1]
    br = 1000
    while n % br:
        br -= 8

    def body(p_ref, w_ref, o_ref):
        o_ref[...] = (
            jnp.dot(p_ref[0], w_ref[:dh], preferred_element_type=jnp.float32)
            + jnp.dot(p_ref[1], w_ref[dh:], preferred_element_type=jnp.float32)
        )

    return pl.pallas_call(
        body,
        grid=(n // br,),
        in_specs=[
            pl.BlockSpec((_NC, br, dh), lambda i: (0, i, 0)),
            pl.BlockSpec((2 * dh, dout), lambda i: (0, 0)),
        ],
        out_specs=pl.BlockSpec((br, dout), lambda i: (i, 0)),
        out_shape=jax.ShapeDtypeStruct((n, dout), jnp.float32),
    )(agg, w)


def kernel(x, edge_index, W):
    n, d = x.shape
    dh = d // 2
    e = edge_index.shape[1]
    cpt = e // (_NS * _CHUNK)
    dst = edge_index[0].astype(jnp.int32)
    src = edge_index[1].astype(jnp.int32)
    x01 = x.reshape(2 * n, dh)
    src_t = src.reshape(_NS, cpt, _CHUNK)
    dst_t = dst.reshape(_NS, cpt, _CHUNK)
    agg = _sc_aggregate(x01, src_t, dst_t, n, dh)
    return _tc_finish(agg, W, n)
